# Initial kernel scaffold; baseline (speedup 1.0000x reference)
import jax
import jax.numpy as jnp
from jax.experimental import pallas as pl


def kernel(x, W):
    raise NotImplementedError("write your pallas kernel here")


# speedup vs baseline: 1.7766x; 1.7766x over previous
import functools

import jax
import jax.numpy as jnp
from jax.experimental import pallas as pl

_NUM_EXPERTS = 64
_BLOCK_TOKENS = 1024


def _router_kernel(x_ref, wt_ref, probs_ref, w_ref, i_ref):
    logits = jnp.dot(x_ref[...], wt_ref[...], preferred_element_type=jnp.float32)
    m1 = jnp.max(logits, axis=-1, keepdims=True)
    e = jnp.exp(logits - m1)
    s = jnp.sum(e, axis=-1, keepdims=True)
    probs_ref[...] = e / s

    iota = jax.lax.broadcasted_iota(jnp.int32, logits.shape, 1)
    i1 = jnp.argmax(logits, axis=-1)
    masked = jnp.where(iota == i1[:, None], -jnp.inf, logits)
    m2 = jnp.max(masked, axis=-1)
    i2 = jnp.argmax(masked, axis=-1)

    r = jnp.exp(m2 - m1[:, 0])
    w1 = 1.0 / (1.0 + r)
    w2 = r / (1.0 + r)
    w_ref[...] = jnp.stack([w1, w2], axis=-1)
    i_ref[...] = jnp.stack([i1, i2], axis=-1).astype(jnp.int32)


@jax.jit
def kernel(x, W):
    b, s, d = x.shape
    n_tokens = b * s
    x2 = x.reshape(n_tokens, d)
    wt = W.T
    grid = (n_tokens // _BLOCK_TOKENS,)
    probs, weights, indices = pl.pallas_call(
        _router_kernel,
        grid=grid,
        in_specs=[
            pl.BlockSpec((_BLOCK_TOKENS, d), lambda i: (i, 0)),
            pl.BlockSpec((d, _NUM_EXPERTS), lambda i: (0, 0)),
        ],
        out_specs=[
            pl.BlockSpec((_BLOCK_TOKENS, _NUM_EXPERTS), lambda i: (i, 0)),
            pl.BlockSpec((_BLOCK_TOKENS, 2), lambda i: (i, 0)),
            pl.BlockSpec((_BLOCK_TOKENS, 2), lambda i: (i, 0)),
        ],
        out_shape=[
            jax.ShapeDtypeStruct((n_tokens, _NUM_EXPERTS), jnp.float32),
            jax.ShapeDtypeStruct((n_tokens, 2), jnp.float32),
            jax.ShapeDtypeStruct((n_tokens, 2), jnp.int32),
        ],
    )(x2, wt)
    return (
        weights.reshape(b, s, 2),
        indices.reshape(b, s, 2),
        probs.reshape(b, s, _NUM_EXPERTS),
    )


# speedup vs baseline: 1.9070x; 1.0734x over previous
import functools

import jax
import jax.numpy as jnp
from jax.experimental import pallas as pl

_NUM_EXPERTS = 64
_BLOCK_TOKENS = 2048


def _router_kernel(x_ref, wt_ref, probs_ref, w_ref, i_ref):
    logits = jnp.dot(x_ref[...], wt_ref[...], preferred_element_type=jnp.float32)
    m1 = jnp.max(logits, axis=-1, keepdims=True)
    e = jnp.exp(logits - m1)
    s = jnp.sum(e, axis=-1, keepdims=True)
    probs_ref[...] = e / s

    iota = jax.lax.broadcasted_iota(jnp.int32, logits.shape, 1)
    i1 = jnp.argmax(logits, axis=-1)
    masked = jnp.where(iota == i1[:, None], -jnp.inf, logits)
    m2 = jnp.max(masked, axis=-1)
    i2 = jnp.argmax(masked, axis=-1)

    r = jnp.exp(m2 - m1[:, 0])
    w1 = 1.0 / (1.0 + r)
    w2 = r / (1.0 + r)
    w_ref[...] = jnp.stack([w1, w2], axis=-1)
    i_ref[...] = jnp.stack([i1, i2], axis=-1).astype(jnp.int32)


@jax.jit
def kernel(x, W):
    b, s, d = x.shape
    n_tokens = b * s
    x2 = x.reshape(n_tokens, d)
    wt = W.T
    grid = (n_tokens // _BLOCK_TOKENS,)
    probs, weights, indices = pl.pallas_call(
        _router_kernel,
        grid=grid,
        in_specs=[
            pl.BlockSpec((_BLOCK_TOKENS, d), lambda i: (i, 0)),
            pl.BlockSpec((d, _NUM_EXPERTS), lambda i: (0, 0)),
        ],
        out_specs=[
            pl.BlockSpec((_BLOCK_TOKENS, _NUM_EXPERTS), lambda i: (i, 0)),
            pl.BlockSpec((_BLOCK_TOKENS, 2), lambda i: (i, 0)),
            pl.BlockSpec((_BLOCK_TOKENS, 2), lambda i: (i, 0)),
        ],
        out_shape=[
            jax.ShapeDtypeStruct((n_tokens, _NUM_EXPERTS), jnp.float32),
            jax.ShapeDtypeStruct((n_tokens, 2), jnp.float32),
            jax.ShapeDtypeStruct((n_tokens, 2), jnp.int32),
        ],
    )(x2, wt)
    return (
        weights.reshape(b, s, 2),
        indices.reshape(b, s, 2),
        probs.reshape(b, s, _NUM_EXPERTS),
    )


# speedup vs baseline: 2.0029x; 1.0503x over previous
import functools

import jax
import jax.numpy as jnp
from jax.experimental import pallas as pl

_NUM_EXPERTS = 64
_BLOCK_TOKENS = 4096


def _router_kernel(x_ref, wt_ref, probs_ref, w_ref, i_ref):
    logits = jnp.dot(x_ref[...], wt_ref[...], preferred_element_type=jnp.float32)
    m1 = jnp.max(logits, axis=-1, keepdims=True)
    e = jnp.exp(logits - m1)
    s = jnp.sum(e, axis=-1, keepdims=True)
    probs_ref[...] = e / s

    iota = jax.lax.broadcasted_iota(jnp.int32, logits.shape, 1)
    i1 = jnp.argmax(logits, axis=-1)
    masked = jnp.where(iota == i1[:, None], -jnp.inf, logits)
    m2 = jnp.max(masked, axis=-1)
    i2 = jnp.argmax(masked, axis=-1)

    r = jnp.exp(m2 - m1[:, 0])
    w1 = 1.0 / (1.0 + r)
    w2 = r / (1.0 + r)
    w_ref[...] = jnp.stack([w1, w2], axis=-1)
    i_ref[...] = jnp.stack([i1, i2], axis=-1).astype(jnp.int32)


@jax.jit
def kernel(x, W):
    b, s, d = x.shape
    n_tokens = b * s
    x2 = x.reshape(n_tokens, d)
    wt = W.T
    grid = (n_tokens // _BLOCK_TOKENS,)
    probs, weights, indices = pl.pallas_call(
        _router_kernel,
        grid=grid,
        in_specs=[
            pl.BlockSpec((_BLOCK_TOKENS, d), lambda i: (i, 0)),
            pl.BlockSpec((d, _NUM_EXPERTS), lambda i: (0, 0)),
        ],
        out_specs=[
            pl.BlockSpec((_BLOCK_TOKENS, _NUM_EXPERTS), lambda i: (i, 0)),
            pl.BlockSpec((_BLOCK_TOKENS, 2), lambda i: (i, 0)),
            pl.BlockSpec((_BLOCK_TOKENS, 2), lambda i: (i, 0)),
        ],
        out_shape=[
            jax.ShapeDtypeStruct((n_tokens, _NUM_EXPERTS), jnp.float32),
            jax.ShapeDtypeStruct((n_tokens, 2), jnp.float32),
            jax.ShapeDtypeStruct((n_tokens, 2), jnp.int32),
        ],
    )(x2, wt)
    return (
        weights.reshape(b, s, 2),
        indices.reshape(b, s, 2),
        probs.reshape(b, s, _NUM_EXPERTS),
    )


# speedup vs baseline: 2.0058x; 1.0015x over previous
import functools

import jax
import jax.numpy as jnp
from jax.experimental import pallas as pl
from jax.experimental.pallas import tpu as pltpu

_NUM_EXPERTS = 64
_BLOCK_TOKENS = 4096


def _router_kernel(x_ref, wt_ref, probs_ref, w_ref, i_ref):
    logits = jnp.dot(x_ref[...], wt_ref[...], preferred_element_type=jnp.float32)
    m1 = jnp.max(logits, axis=-1, keepdims=True)
    e = jnp.exp(logits - m1)
    s = jnp.sum(e, axis=-1, keepdims=True)
    probs_ref[...] = e / s

    iota = jax.lax.broadcasted_iota(jnp.int32, logits.shape, 1)
    i1 = jnp.argmax(logits, axis=-1)
    masked = jnp.where(iota == i1[:, None], -jnp.inf, logits)
    m2 = jnp.max(masked, axis=-1)
    i2 = jnp.argmax(masked, axis=-1)

    r = jnp.exp(m2 - m1[:, 0])
    w1 = 1.0 / (1.0 + r)
    w2 = r / (1.0 + r)
    w_ref[...] = jnp.stack([w1, w2], axis=-1)
    i_ref[...] = jnp.stack([i1, i2], axis=-1).astype(jnp.int32)


@jax.jit
def kernel(x, W):
    b, s, d = x.shape
    n_tokens = b * s
    x2 = x.reshape(n_tokens, d)
    wt = W.T
    grid = (n_tokens // _BLOCK_TOKENS,)
    probs, weights, indices = pl.pallas_call(
        _router_kernel,
        grid=grid,
        in_specs=[
            pl.BlockSpec((_BLOCK_TOKENS, d), lambda i: (i, 0)),
            pl.BlockSpec((d, _NUM_EXPERTS), lambda i: (0, 0)),
        ],
        out_specs=[
            pl.BlockSpec((_BLOCK_TOKENS, _NUM_EXPERTS), lambda i: (i, 0)),
            pl.BlockSpec((_BLOCK_TOKENS, 2), lambda i: (i, 0)),
            pl.BlockSpec((_BLOCK_TOKENS, 2), lambda i: (i, 0)),
        ],
        out_shape=[
            jax.ShapeDtypeStruct((n_tokens, _NUM_EXPERTS), jnp.float32),
            jax.ShapeDtypeStruct((n_tokens, 2), jnp.float32),
            jax.ShapeDtypeStruct((n_tokens, 2), jnp.int32),
        ],
        compiler_params=pltpu.CompilerParams(
            dimension_semantics=("parallel",),
        ),
    )(x2, wt)
    return (
        weights.reshape(b, s, 2),
        indices.reshape(b, s, 2),
        probs.reshape(b, s, _NUM_EXPERTS),
    )


# speedup vs baseline: 2.2198x; 1.1067x over previous
import jax
import jax.numpy as jnp
from jax.experimental import pallas as pl
from jax.experimental.pallas import tpu as pltpu

_NUM_EXPERTS = 64
_BLOCK_TOKENS = 4096


def _router_kernel(x_ref, w_ref, probs_ref, w_out_ref, i_out_ref):
    x = x_ref[0]
    logits = jax.lax.dot_general(
        x, w_ref[...],
        dimension_numbers=(((1,), (1,)), ((), ())),
        preferred_element_type=jnp.float32,
    )
    m1 = jnp.max(logits, axis=-1, keepdims=True)
    e = jnp.exp(logits - m1)
    s = jnp.sum(e, axis=-1, keepdims=True)
    probs_ref[0] = e / s

    iota = jax.lax.broadcasted_iota(jnp.int32, logits.shape, 1)
    i1 = jnp.argmax(logits, axis=-1)
    masked = jnp.where(iota == i1[:, None], -jnp.inf, logits)
    m2 = jnp.max(masked, axis=-1)
    i2 = jnp.argmax(masked, axis=-1)

    r = jnp.exp(m2 - m1[:, 0])
    w1 = 1.0 / (1.0 + r)
    w2 = r / (1.0 + r)
    w_out_ref[0] = jnp.stack([w1, w2], axis=-1)
    i_out_ref[0] = jnp.stack([i1, i2], axis=-1).astype(jnp.int32)


@jax.jit
def kernel(x, W):
    b, s, d = x.shape
    grid = (b, s // _BLOCK_TOKENS)
    probs, weights, indices = pl.pallas_call(
        _router_kernel,
        grid=grid,
        in_specs=[
            pl.BlockSpec((1, _BLOCK_TOKENS, d), lambda i, j: (i, j, 0)),
            pl.BlockSpec((_NUM_EXPERTS, d), lambda i, j: (0, 0)),
        ],
        out_specs=[
            pl.BlockSpec((1, _BLOCK_TOKENS, _NUM_EXPERTS), lambda i, j: (i, j, 0)),
            pl.BlockSpec((1, _BLOCK_TOKENS, 2), lambda i, j: (i, j, 0)),
            pl.BlockSpec((1, _BLOCK_TOKENS, 2), lambda i, j: (i, j, 0)),
        ],
        out_shape=[
            jax.ShapeDtypeStruct((b, s, _NUM_EXPERTS), jnp.float32),
            jax.ShapeDtypeStruct((b, s, 2), jnp.float32),
            jax.ShapeDtypeStruct((b, s, 2), jnp.int32),
        ],
        compiler_params=pltpu.CompilerParams(
            dimension_semantics=("parallel", "parallel"),
        ),
    )(x, W)
    return (weights, indices, probs)


# speedup vs baseline: 4.8992x; 2.2070x over previous
import jax
import jax.numpy as jnp
from jax.experimental import pallas as pl
from jax.experimental.pallas import tpu as pltpu

_NUM_EXPERTS = 64
_BLOCK_TOKENS = 4096


def _router_kernel(x_ref, w_ref, probs_ref, w_out_ref, i_out_ref):
    x = x_ref[0]
    logits = jax.lax.dot_general(
        w_ref[...], x,
        dimension_numbers=(((1,), (1,)), ((), ())),
        preferred_element_type=jnp.float32,
    )
    m1 = jnp.max(logits, axis=0, keepdims=True)
    e = jnp.exp(logits - m1)
    s = jnp.sum(e, axis=0, keepdims=True)
    probs_ref[0] = e / s

    iota = jax.lax.broadcasted_iota(jnp.int32, logits.shape, 0)
    i1 = jnp.argmax(logits, axis=0)
    masked = jnp.where(iota == i1[None, :], -jnp.inf, logits)
    m2 = jnp.max(masked, axis=0)
    i2 = jnp.argmax(masked, axis=0)

    r = jnp.exp(m2 - m1[0])
    w1 = 1.0 / (1.0 + r)
    w2 = r / (1.0 + r)
    w_out_ref[0] = jnp.stack([w1, w2], axis=0)
    i_out_ref[0] = jnp.stack([i1, i2], axis=0).astype(jnp.int32)


@jax.jit
def kernel(x, W):
    b, s, d = x.shape
    grid = (b, s // _BLOCK_TOKENS)
    probs_t, weights_t, indices_t = pl.pallas_call(
        _router_kernel,
        grid=grid,
        in_specs=[
            pl.BlockSpec((1, _BLOCK_TOKENS, d), lambda i, j: (i, j, 0)),
            pl.BlockSpec((_NUM_EXPERTS, d), lambda i, j: (0, 0)),
        ],
        out_specs=[
            pl.BlockSpec((1, _NUM_EXPERTS, _BLOCK_TOKENS), lambda i, j: (i, 0, j)),
            pl.BlockSpec((1, 2, _BLOCK_TOKENS), lambda i, j: (i, 0, j)),
            pl.BlockSpec((1, 2, _BLOCK_TOKENS), lambda i, j: (i, 0, j)),
        ],
        out_shape=[
            jax.ShapeDtypeStruct((b, _NUM_EXPERTS, s), jnp.float32),
            jax.ShapeDtypeStruct((b, 2, s), jnp.float32),
            jax.ShapeDtypeStruct((b, 2, s), jnp.int32),
        ],
        compiler_params=pltpu.CompilerParams(
            dimension_semantics=("parallel", "parallel"),
        ),
    )(x, W)
    return (
        jnp.transpose(weights_t, (0, 2, 1)),
        jnp.transpose(indices_t, (0, 2, 1)),
        jnp.transpose(probs_t, (0, 2, 1)),
    )
